# baseline (speedup 1.0000x reference)
import functools

import jax
import jax.numpy as jnp
from jax import lax
from jax.experimental import pallas as pl
from jax.experimental.pallas import tpu as pltpu
from jax.experimental.pallas import tpu_sc as plsc

_N = 10000
_NP = 10240
_D = 128
_E = 320000
_NC = 2
_NS = 16
_NW = _NC * _NS
_EPT = _E // _NW
_CH = 80
_NCH = _EPT // _CH
_RPT = _NP // _NS
_RC = 80
_NRC = _RPT // _RC
_DW = 16
_L = 16

_mesh = plsc.VectorSubcoreMesh(core_axis_name="c", subcore_axis_name="s")


def _zero_rows(ref, nrows, width):
  z = jnp.zeros((_L,), jnp.float32)

  @pl.loop(0, nrows)
  def _(r):
    for p in range(width // _L):
      ref[r, pl.ds(p * _L, _L)] = z


def _fill_ones(ref, nrows, width):
  o = jnp.ones((_L,), jnp.float32)

  @pl.loop(0, nrows)
  def _(r):
    for p in range(width // _L):
      ref[r, pl.ds(p * _L, _L)] = o


def _segsum_body(x_hbm, src_hbm, dst_hbm, out_hbm, idx_s, idxd, rows, acc):
  c = lax.axis_index("c")
  s = lax.axis_index("s")
  w = c * _NS + s

  pltpu.sync_copy(src_hbm.at[w], idx_s)

  _zero_rows(rows, _CH, _D)
  for k in range(_NRC):
    pltpu.sync_copy(rows, acc.at[pl.ds(s * _RPT + k * _RC, _RC)])
  plsc.subcore_barrier()

  @pl.loop(0, _NCH)
  def _(j):
    pltpu.sync_copy(dst_hbm.at[w, j], idxd)
    pltpu.sync_copy(x_hbm.at[idx_s.at[j]], rows)
    pltpu.sync_copy(rows, acc.at[idxd], add=True)

  plsc.subcore_barrier()

  for k in range(_NRC):
    r0 = s * _RPT + k * _RC
    pltpu.sync_copy(acc.at[pl.ds(r0, _RC)], out_hbm.at[c, pl.ds(r0, _RC)])


_segsum = pl.kernel(
    _segsum_body,
    out_type=jax.ShapeDtypeStruct((_NC, _NP, _D), jnp.float32),
    mesh=_mesh,
    scratch_types=[
        pltpu.VMEM((_NCH, _CH), jnp.int32),
        pltpu.VMEM((_CH,), jnp.int32),
        pltpu.VMEM((_CH, _D), jnp.float32),
        pltpu.VMEM_SHARED((_NP, _D), jnp.float32),
    ],
)


def _deg_body(dst_hbm, deg_hbm, idxd, buf, dacc):
  c = lax.axis_index("c")
  s = lax.axis_index("s")
  w = c * _NS + s

  _zero_rows(buf, _CH, _D)
  for k in range(_NRC):
    pltpu.sync_copy(buf, dacc.at[pl.ds(s * _RPT + k * _RC, _RC)])
  _fill_ones(buf, _CH, _D)
  plsc.subcore_barrier()

  @pl.loop(0, _NCH)
  def _(j):
    pltpu.sync_copy(dst_hbm.at[w, j], idxd)
    pltpu.sync_copy(buf, dacc.at[idxd], add=True)

  plsc.subcore_barrier()
  for k in range(_NRC):
    r0 = s * _RPT + k * _RC
    pltpu.sync_copy(dacc.at[pl.ds(r0, _RC)], deg_hbm.at[c, pl.ds(r0, _RC)])


_deg = pl.kernel(
    _deg_body,
    out_type=jax.ShapeDtypeStruct((_NC, _NP, _D), jnp.float32),
    mesh=_mesh,
    scratch_types=[
        pltpu.VMEM((_CH,), jnp.int32),
        pltpu.VMEM((_CH, _D), jnp.float32),
        pltpu.VMEM_SHARED((_NP, _D), jnp.float32),
    ],
)


def _dense_body(relu, p_ref, d_ref, x_ref, wn_ref, ws_ref, b_ref, o_ref):
  agg = p_ref[0] + p_ref[1]
  deg = d_ref[0, :, 0:1] + d_ref[1, :, 0:1]
  mean = agg / jnp.maximum(deg, 1.0)
  h = (jnp.dot(mean, wn_ref[...], preferred_element_type=jnp.float32)
       + jnp.dot(x_ref[...], ws_ref[...], preferred_element_type=jnp.float32)
       + b_ref[...])
  o_ref[...] = jnp.maximum(h, 0.0) if relu else h


_BR = 1024


def _make_dense(relu):
  return pl.pallas_call(
      functools.partial(_dense_body, relu),
      grid=(_NP // _BR,),
      in_specs=[
          pl.BlockSpec((_NC, _BR, _D), lambda i: (0, i, 0)),
          pl.BlockSpec((_NC, _BR, _D), lambda i: (0, i, 0)),
          pl.BlockSpec((_BR, _D), lambda i: (i, 0)),
          pl.BlockSpec((_D, _D), lambda i: (0, 0)),
          pl.BlockSpec((_D, _D), lambda i: (0, 0)),
          pl.BlockSpec((1, _D), lambda i: (0, 0)),
      ],
      out_specs=pl.BlockSpec((_BR, _D), lambda i: (i, 0)),
      out_shape=jax.ShapeDtypeStruct((_NP, _D), jnp.float32),
  )


_dense_relu = _make_dense(True)
_dense = _make_dense(False)


def _scores_body(h_hbm, si_hbm, di_hbm, nsi_hbm, ndi_hbm,
                 pos_hbm, neg_hbm, ia0, ib0, ia1, ib1,
                 hs0, hd0, hs1, hd1, sbuf, semi0, semi1, semg0, semg1, h2s):
  c = lax.axis_index("c")
  s = lax.axis_index("s")
  w = c * _NS + s
  lanes = lax.iota(jnp.int32, _L)

  for k in range(_NRC):
    r0 = s * _RPT + k * _RC
    pltpu.sync_copy(h_hbm.at[pl.ds(r0, _RC)], hs0)
    pltpu.sync_copy(hs0, h2s.at[pl.ds(r0, _RC)])
  plsc.subcore_barrier()

  def start_idx(a_h, b_h, j, ia, ib, sem):
    pltpu.async_copy(a_h.at[pl.ds(w * _EPT + j * _CH, _CH)], ia, sem)
    pltpu.async_copy(b_h.at[pl.ds(w * _EPT + j * _CH, _CH)], ib, sem)

  def wait_idx(a_h, b_h, j, ia, ib, sem):
    pltpu.make_async_copy(a_h.at[pl.ds(w * _EPT + j * _CH, _CH)], ia, sem).wait()
    pltpu.make_async_copy(b_h.at[pl.ds(w * _EPT + j * _CH, _CH)], ib, sem).wait()

  def start_gather(ia, ib, hs, hd, sem):
    pltpu.async_copy(h2s.at[ia], hs, sem)
    pltpu.async_copy(h2s.at[ib], hd, sem)

  def wait_gather(ia, ib, hs, hd, sem):
    pltpu.make_async_copy(h2s.at[ia], hs, sem).wait()
    pltpu.make_async_copy(h2s.at[ib], hd, sem).wait()

  def compute(j, hs, hd, out_h):
    @pl.loop(0, _CH // _L)
    def _(g):
      erow = g * _L + lanes
      zero = jnp.zeros((_L,), jnp.float32)

      @pl.loop(0, _D // 32, init_carry=(zero,) * 8)
      def accs(t, carry):
        new = list(carry)
        base = t * 32
        for u in range(32):
          col = base + jnp.full((_L,), u, jnp.int32)
          new[u % 8] = new[u % 8] + (plsc.load_gather(hs, (erow, col))
                                     * plsc.load_gather(hd, (erow, col)))
        return tuple(new)

      acc = (((accs[0] + accs[1]) + (accs[2] + accs[3]))
             + ((accs[4] + accs[5]) + (accs[6] + accs[7])))
      sbuf[pl.ds(g * _L, _L)] = acc

    pltpu.sync_copy(sbuf, out_h.at[pl.ds(w * _EPT + j * _CH, _CH)])

  for (a_h, b_h, out_h) in ((si_hbm, di_hbm, pos_hbm),
                            (nsi_hbm, ndi_hbm, neg_hbm)):
    start_idx(a_h, b_h, 0, ia0, ib0, semi0)
    wait_idx(a_h, b_h, 0, ia0, ib0, semi0)
    start_gather(ia0, ib0, hs0, hd0, semg0)
    start_idx(a_h, b_h, 1, ia1, ib1, semi1)

    @pl.loop(0, (_NCH - 1) // 2)
    def _(j2):
      j = 2 * j2
      wait_gather(ia0, ib0, hs0, hd0, semg0)
      start_idx(a_h, b_h, j + 2, ia0, ib0, semi0)
      wait_idx(a_h, b_h, j + 1, ia1, ib1, semi1)
      start_gather(ia1, ib1, hs1, hd1, semg1)
      compute(j, hs0, hd0, out_h)
      wait_gather(ia1, ib1, hs1, hd1, semg1)
      start_idx(a_h, b_h, jnp.minimum(j + 3, _NCH - 1), ia1, ib1, semi1)
      wait_idx(a_h, b_h, j + 2, ia0, ib0, semi0)
      start_gather(ia0, ib0, hs0, hd0, semg0)
      compute(j + 1, hs1, hd1, out_h)

    wait_gather(ia0, ib0, hs0, hd0, semg0)
    wait_idx(a_h, b_h, _NCH - 1, ia1, ib1, semi1)
    compute(_NCH - 1, hs0, hd0, out_h)


_scores = pl.kernel(
    _scores_body,
    compiler_params=pltpu.CompilerParams(needs_layout_passes=False),
    out_type=(jax.ShapeDtypeStruct((_E,), jnp.float32),
              jax.ShapeDtypeStruct((_E,), jnp.float32)),
    mesh=_mesh,
    scratch_types=[
        pltpu.VMEM((_CH,), jnp.int32),
        pltpu.VMEM((_CH,), jnp.int32),
        pltpu.VMEM((_CH,), jnp.int32),
        pltpu.VMEM((_CH,), jnp.int32),
        pltpu.VMEM((_CH, _D), jnp.float32),
        pltpu.VMEM((_CH, _D), jnp.float32),
        pltpu.VMEM((_CH, _D), jnp.float32),
        pltpu.VMEM((_CH, _D), jnp.float32),
        pltpu.VMEM((_CH,), jnp.float32),
        pltpu.SemaphoreType.DMA,
        pltpu.SemaphoreType.DMA,
        pltpu.SemaphoreType.DMA,
        pltpu.SemaphoreType.DMA,
        pltpu.VMEM_SHARED((_NP, _D), jnp.float32),
    ],
)


def kernel(x, edge_index, neg_edge_index, W_neigh1, W_self1, b1,
           W_neigh2, W_self2, b2):
  src = edge_index[0].reshape(_NW, _NCH, _CH)
  dst = edge_index[1].reshape(_NW, _NCH, _CH)
  nsrc = neg_edge_index[0].reshape(_NW, _NCH, _CH)
  ndst = neg_edge_index[1].reshape(_NW, _NCH, _CH)

  xp = jnp.pad(x, ((0, _NP - _N), (0, 0)))
  degp = _deg(dst)
  p1 = _segsum(xp, src, dst)
  h1 = _dense_relu(p1, degp, xp, W_neigh1, W_self1, b1.reshape(1, _D))
  p2 = _segsum(h1, src, dst)
  h2 = _dense(p2, degp, h1, W_neigh2, W_self2, b2.reshape(1, _D))
  pos, neg = _scores(h2, edge_index[0], edge_index[1],
                     neg_edge_index[0], neg_edge_index[1])
  return pos.reshape(_E, 1), neg.reshape(_E, 1)


# speedup vs baseline: 2.8640x; 2.8640x over previous
import functools

import jax
import jax.numpy as jnp
from jax import lax
from jax.experimental import pallas as pl
from jax.experimental.pallas import tpu as pltpu
from jax.experimental.pallas import tpu_sc as plsc

_N = 10000
_NP = 10240
_D = 128
_E = 320000
_NC = 2
_NS = 16
_NW = _NC * _NS
_EPT = _E // _NW
_CH = 80
_NCH = _EPT // _CH
_RPT = _NP // _NS
_RC = 80
_NRC = _RPT // _RC
_DW = 16
_L = 16

_mesh = plsc.VectorSubcoreMesh(core_axis_name="c", subcore_axis_name="s")


def _zero_rows(ref, nrows, width):
  z = jnp.zeros((_L,), jnp.float32)

  @pl.loop(0, nrows)
  def _(r):
    for p in range(width // _L):
      ref[r, pl.ds(p * _L, _L)] = z


def _fill_ones(ref, nrows, width):
  o = jnp.ones((_L,), jnp.float32)

  @pl.loop(0, nrows)
  def _(r):
    for p in range(width // _L):
      ref[r, pl.ds(p * _L, _L)] = o


def _segsum_body(x_hbm, src_hbm, dst_hbm, out_hbm, idx_s, idxd, rows, acc):
  c = lax.axis_index("c")
  s = lax.axis_index("s")
  w = c * _NS + s

  pltpu.sync_copy(src_hbm.at[w], idx_s)

  _zero_rows(rows, _CH, _D)
  for k in range(_NRC):
    pltpu.sync_copy(rows, acc.at[pl.ds(s * _RPT + k * _RC, _RC)])
  plsc.subcore_barrier()

  @pl.loop(0, _NCH)
  def _(j):
    pltpu.sync_copy(dst_hbm.at[w, j], idxd)
    pltpu.sync_copy(x_hbm.at[idx_s.at[j]], rows)
    pltpu.sync_copy(rows, acc.at[idxd], add=True)

  plsc.subcore_barrier()

  for k in range(_NRC):
    r0 = s * _RPT + k * _RC
    pltpu.sync_copy(acc.at[pl.ds(r0, _RC)], out_hbm.at[c, pl.ds(r0, _RC)])


_segsum = pl.kernel(
    _segsum_body,
    out_type=jax.ShapeDtypeStruct((_NC, _NP, _D), jnp.float32),
    mesh=_mesh,
    scratch_types=[
        pltpu.VMEM((_NCH, _CH), jnp.int32),
        pltpu.VMEM((_CH,), jnp.int32),
        pltpu.VMEM((_CH, _D), jnp.float32),
        pltpu.VMEM_SHARED((_NP, _D), jnp.float32),
    ],
)


def _deg_body(dst_hbm, deg_hbm, idxd, buf, dacc):
  c = lax.axis_index("c")
  s = lax.axis_index("s")
  w = c * _NS + s

  _zero_rows(buf, _CH, _D)
  for k in range(_NRC):
    pltpu.sync_copy(buf, dacc.at[pl.ds(s * _RPT + k * _RC, _RC)])
  _fill_ones(buf, _CH, _D)
  plsc.subcore_barrier()

  @pl.loop(0, _NCH)
  def _(j):
    pltpu.sync_copy(dst_hbm.at[w, j], idxd)
    pltpu.sync_copy(buf, dacc.at[idxd], add=True)

  plsc.subcore_barrier()
  for k in range(_NRC):
    r0 = s * _RPT + k * _RC
    pltpu.sync_copy(dacc.at[pl.ds(r0, _RC)], deg_hbm.at[c, pl.ds(r0, _RC)])


_deg = pl.kernel(
    _deg_body,
    out_type=jax.ShapeDtypeStruct((_NC, _NP, _D), jnp.float32),
    mesh=_mesh,
    scratch_types=[
        pltpu.VMEM((_CH,), jnp.int32),
        pltpu.VMEM((_CH, _D), jnp.float32),
        pltpu.VMEM_SHARED((_NP, _D), jnp.float32),
    ],
)


def _dense_body(relu, p_ref, d_ref, x_ref, wn_ref, ws_ref, b_ref, o_ref):
  agg = p_ref[0] + p_ref[1]
  deg = d_ref[0, :, 0:1] + d_ref[1, :, 0:1]
  mean = agg / jnp.maximum(deg, 1.0)
  h = (jnp.dot(mean, wn_ref[...], preferred_element_type=jnp.float32)
       + jnp.dot(x_ref[...], ws_ref[...], preferred_element_type=jnp.float32)
       + b_ref[...])
  o_ref[...] = jnp.maximum(h, 0.0) if relu else h


_BR = 1024


def _make_dense(relu):
  return pl.pallas_call(
      functools.partial(_dense_body, relu),
      grid=(_NP // _BR,),
      in_specs=[
          pl.BlockSpec((_NC, _BR, _D), lambda i: (0, i, 0)),
          pl.BlockSpec((_NC, _BR, _D), lambda i: (0, i, 0)),
          pl.BlockSpec((_BR, _D), lambda i: (i, 0)),
          pl.BlockSpec((_D, _D), lambda i: (0, 0)),
          pl.BlockSpec((_D, _D), lambda i: (0, 0)),
          pl.BlockSpec((1, _D), lambda i: (0, 0)),
      ],
      out_specs=pl.BlockSpec((_BR, _D), lambda i: (i, 0)),
      out_shape=jax.ShapeDtypeStruct((_NP, _D), jnp.float32),
  )


_dense_relu = _make_dense(True)
_dense = _make_dense(False)


def _scores_body(h_hbm, si_hbm, di_hbm, nsi_hbm, ndi_hbm,
                 pos_hbm, neg_hbm, ia0, ib0, ia1, ib1,
                 hs0, hd0, hs1, hd1, sbuf, semi0, semi1, semg0, semg1, h2s):
  c = lax.axis_index("c")
  s = lax.axis_index("s")
  w = c * _NS + s
  lanes = lax.iota(jnp.int32, _L)

  for k in range(_NRC):
    r0 = s * _RPT + k * _RC
    pltpu.sync_copy(h_hbm.at[pl.ds(r0, _RC)], hs0)
    pltpu.sync_copy(hs0, h2s.at[pl.ds(r0, _RC)])
  plsc.subcore_barrier()

  def start_idx(a_h, b_h, j, ia, ib, sem):
    pltpu.async_copy(a_h.at[pl.ds(w * _EPT + j * _CH, _CH)], ia, sem)
    pltpu.async_copy(b_h.at[pl.ds(w * _EPT + j * _CH, _CH)], ib, sem)

  def wait_idx(a_h, b_h, j, ia, ib, sem):
    pltpu.make_async_copy(a_h.at[pl.ds(w * _EPT + j * _CH, _CH)], ia, sem).wait()
    pltpu.make_async_copy(b_h.at[pl.ds(w * _EPT + j * _CH, _CH)], ib, sem).wait()

  def start_gather(ia, ib, hs, hd, sem):
    pltpu.async_copy(h2s.at[ia], hs, sem)
    pltpu.async_copy(h2s.at[ib], hd, sem)

  def wait_gather(ia, ib, hs, hd, sem):
    pltpu.make_async_copy(h2s.at[ia], hs, sem).wait()
    pltpu.make_async_copy(h2s.at[ib], hd, sem).wait()

  def compute(j, hs, hd, out_h):
    @pl.loop(0, _CH // _L)
    def _(g):
      erow = g * _L + lanes
      zero = jnp.zeros((_L,), jnp.float32)

      @pl.loop(0, _D // 32, init_carry=(zero,) * 8)
      def accs(t, carry):
        new = list(carry)
        base = t * 32
        for u in range(32):
          col = jnp.bitwise_and(base + (u + lanes), _D - 1)
          new[u % 8] = new[u % 8] + (plsc.load_gather(hs, (erow, col))
                                     * plsc.load_gather(hd, (erow, col)))
        return tuple(new)

      acc = (((accs[0] + accs[1]) + (accs[2] + accs[3]))
             + ((accs[4] + accs[5]) + (accs[6] + accs[7])))
      sbuf[pl.ds(g * _L, _L)] = acc

    pltpu.sync_copy(sbuf, out_h.at[pl.ds(w * _EPT + j * _CH, _CH)])

  for (a_h, b_h, out_h) in ((si_hbm, di_hbm, pos_hbm),
                            (nsi_hbm, ndi_hbm, neg_hbm)):
    start_idx(a_h, b_h, 0, ia0, ib0, semi0)
    wait_idx(a_h, b_h, 0, ia0, ib0, semi0)
    start_gather(ia0, ib0, hs0, hd0, semg0)
    start_idx(a_h, b_h, 1, ia1, ib1, semi1)

    @pl.loop(0, (_NCH - 1) // 2)
    def _(j2):
      j = 2 * j2
      wait_gather(ia0, ib0, hs0, hd0, semg0)
      start_idx(a_h, b_h, j + 2, ia0, ib0, semi0)
      wait_idx(a_h, b_h, j + 1, ia1, ib1, semi1)
      start_gather(ia1, ib1, hs1, hd1, semg1)
      compute(j, hs0, hd0, out_h)
      wait_gather(ia1, ib1, hs1, hd1, semg1)
      start_idx(a_h, b_h, jnp.minimum(j + 3, _NCH - 1), ia1, ib1, semi1)
      wait_idx(a_h, b_h, j + 2, ia0, ib0, semi0)
      start_gather(ia0, ib0, hs0, hd0, semg0)
      compute(j + 1, hs1, hd1, out_h)

    wait_gather(ia0, ib0, hs0, hd0, semg0)
    wait_idx(a_h, b_h, _NCH - 1, ia1, ib1, semi1)
    compute(_NCH - 1, hs0, hd0, out_h)


_scores = pl.kernel(
    _scores_body,
    compiler_params=pltpu.CompilerParams(needs_layout_passes=False),
    out_type=(jax.ShapeDtypeStruct((_E,), jnp.float32),
              jax.ShapeDtypeStruct((_E,), jnp.float32)),
    mesh=_mesh,
    scratch_types=[
        pltpu.VMEM((_CH,), jnp.int32),
        pltpu.VMEM((_CH,), jnp.int32),
        pltpu.VMEM((_CH,), jnp.int32),
        pltpu.VMEM((_CH,), jnp.int32),
        pltpu.VMEM((_CH, _D), jnp.float32),
        pltpu.VMEM((_CH, _D), jnp.float32),
        pltpu.VMEM((_CH, _D), jnp.float32),
        pltpu.VMEM((_CH, _D), jnp.float32),
        pltpu.VMEM((_CH,), jnp.float32),
        pltpu.SemaphoreType.DMA,
        pltpu.SemaphoreType.DMA,
        pltpu.SemaphoreType.DMA,
        pltpu.SemaphoreType.DMA,
        pltpu.VMEM_SHARED((_NP, _D), jnp.float32),
    ],
)


def kernel(x, edge_index, neg_edge_index, W_neigh1, W_self1, b1,
           W_neigh2, W_self2, b2):
  src = edge_index[0].reshape(_NW, _NCH, _CH)
  dst = edge_index[1].reshape(_NW, _NCH, _CH)
  nsrc = neg_edge_index[0].reshape(_NW, _NCH, _CH)
  ndst = neg_edge_index[1].reshape(_NW, _NCH, _CH)

  xp = jnp.pad(x, ((0, _NP - _N), (0, 0)))
  degp = _deg(dst)
  p1 = _segsum(xp, src, dst)
  h1 = _dense_relu(p1, degp, xp, W_neigh1, W_self1, b1.reshape(1, _D))
  p2 = _segsum(h1, src, dst)
  h2 = _dense(p2, degp, h1, W_neigh2, W_self2, b2.reshape(1, _D))
  pos, neg = _scores(h2, edge_index[0], edge_index[1],
                     neg_edge_index[0], neg_edge_index[1])
  return pos.reshape(_E, 1), neg.reshape(_E, 1)


# speedup vs baseline: 3.1801x; 1.1104x over previous
import functools

import jax
import jax.numpy as jnp
from jax import lax
from jax.experimental import pallas as pl
from jax.experimental.pallas import tpu as pltpu
from jax.experimental.pallas import tpu_sc as plsc

_N = 10000
_NP = 10240
_D = 128
_E = 320000
_NC = 2
_NS = 16
_NW = _NC * _NS
_EPT = _E // _NW
_CH = 80
_NCH = _EPT // _CH
_RPT = _NP // _NS
_RC = 80
_NRC = _RPT // _RC
_DW = 16
_L = 16

_mesh = plsc.VectorSubcoreMesh(core_axis_name="c", subcore_axis_name="s")


def _zero_rows(ref, nrows, width):
  z = jnp.zeros((_L,), jnp.float32)

  @pl.loop(0, nrows)
  def _(r):
    for p in range(width // _L):
      ref[r, pl.ds(p * _L, _L)] = z


def _fill_ones(ref, nrows, width):
  o = jnp.ones((_L,), jnp.float32)

  @pl.loop(0, nrows)
  def _(r):
    for p in range(width // _L):
      ref[r, pl.ds(p * _L, _L)] = o


def _segsum_body(x_hbm, src_hbm, dst_hbm, out_hbm,
                 idx_s, idxd0, idxd1, rows0, rows1,
                 semg0, semg1, sems0, sems1, acc):
  c = lax.axis_index("c")
  s = lax.axis_index("s")
  w = c * _NS + s

  pltpu.sync_copy(src_hbm.at[w], idx_s)

  _zero_rows(rows0, _CH, _D)
  for k in range(_NRC):
    pltpu.sync_copy(rows0, acc.at[pl.ds(s * _RPT + k * _RC, _RC)])
  plsc.subcore_barrier()

  def load_idxd(j, idxd):
    pltpu.sync_copy(dst_hbm.at[w, j], idxd)

  def start_gather(j, rows, sem):
    pltpu.async_copy(x_hbm.at[idx_s.at[j]], rows, sem)

  def wait_gather(j, rows, sem):
    pltpu.make_async_copy(x_hbm.at[idx_s.at[j]], rows, sem).wait()

  def start_scatter(rows, idxd, sem):
    pltpu.async_copy(rows, acc.at[idxd], sem, add=True)

  def wait_scatter(rows, idxd, sem):
    pltpu.make_async_copy(rows, acc.at[idxd], sem).wait()

  load_idxd(0, idxd0)
  start_gather(0, rows0, semg0)

  @pl.loop(0, (_NCH - 1) // 2)
  def _(j2):
    j = 2 * j2
    wait_gather(j, rows0, semg0)
    start_scatter(rows0, idxd0, sems0)

    @pl.when(j2 > 0)
    def _():
      wait_scatter(rows1, idxd1, sems1)
    load_idxd(j + 1, idxd1)
    start_gather(j + 1, rows1, semg1)
    wait_gather(j + 1, rows1, semg1)
    start_scatter(rows1, idxd1, sems1)
    wait_scatter(rows0, idxd0, sems0)
    load_idxd(j + 2, idxd0)
    start_gather(j + 2, rows0, semg0)

  wait_gather(_NCH - 1, rows0, semg0)
  start_scatter(rows0, idxd0, sems0)
  wait_scatter(rows1, idxd1, sems1)
  wait_scatter(rows0, idxd0, sems0)

  plsc.subcore_barrier()

  for k in range(_NRC):
    r0 = s * _RPT + k * _RC
    pltpu.sync_copy(acc.at[pl.ds(r0, _RC)], out_hbm.at[c, pl.ds(r0, _RC)])


_segsum = pl.kernel(
    _segsum_body,
    out_type=jax.ShapeDtypeStruct((_NC, _NP, _D), jnp.float32),
    mesh=_mesh,
    scratch_types=[
        pltpu.VMEM((_NCH, _CH), jnp.int32),
        pltpu.VMEM((_CH,), jnp.int32),
        pltpu.VMEM((_CH,), jnp.int32),
        pltpu.VMEM((_CH, _D), jnp.float32),
        pltpu.VMEM((_CH, _D), jnp.float32),
        pltpu.SemaphoreType.DMA,
        pltpu.SemaphoreType.DMA,
        pltpu.SemaphoreType.DMA,
        pltpu.SemaphoreType.DMA,
        pltpu.VMEM_SHARED((_NP, _D), jnp.float32),
    ],
)


def _deg_body(dst_hbm, deg_hbm, idxd, buf, dacc):
  c = lax.axis_index("c")
  s = lax.axis_index("s")
  w = c * _NS + s

  _zero_rows(buf, _CH, _D)
  for k in range(_NRC):
    pltpu.sync_copy(buf, dacc.at[pl.ds(s * _RPT + k * _RC, _RC)])
  _fill_ones(buf, _CH, _D)
  plsc.subcore_barrier()

  @pl.loop(0, _NCH)
  def _(j):
    pltpu.sync_copy(dst_hbm.at[w, j], idxd)
    pltpu.sync_copy(buf, dacc.at[idxd], add=True)

  plsc.subcore_barrier()
  for k in range(_NRC):
    r0 = s * _RPT + k * _RC
    pltpu.sync_copy(dacc.at[pl.ds(r0, _RC)], deg_hbm.at[c, pl.ds(r0, _RC)])


_deg = pl.kernel(
    _deg_body,
    out_type=jax.ShapeDtypeStruct((_NC, _NP, _D), jnp.float32),
    mesh=_mesh,
    scratch_types=[
        pltpu.VMEM((_CH,), jnp.int32),
        pltpu.VMEM((_CH, _D), jnp.float32),
        pltpu.VMEM_SHARED((_NP, _D), jnp.float32),
    ],
)


def _dense_body(relu, p_ref, d_ref, x_ref, wn_ref, ws_ref, b_ref, o_ref):
  agg = p_ref[0] + p_ref[1]
  deg = d_ref[0, :, 0:1] + d_ref[1, :, 0:1]
  mean = agg / jnp.maximum(deg, 1.0)
  h = (jnp.dot(mean, wn_ref[...], preferred_element_type=jnp.float32)
       + jnp.dot(x_ref[...], ws_ref[...], preferred_element_type=jnp.float32)
       + b_ref[...])
  o_ref[...] = jnp.maximum(h, 0.0) if relu else h


_BR = 1024


def _make_dense(relu):
  return pl.pallas_call(
      functools.partial(_dense_body, relu),
      grid=(_NP // _BR,),
      in_specs=[
          pl.BlockSpec((_NC, _BR, _D), lambda i: (0, i, 0)),
          pl.BlockSpec((_NC, _BR, _D), lambda i: (0, i, 0)),
          pl.BlockSpec((_BR, _D), lambda i: (i, 0)),
          pl.BlockSpec((_D, _D), lambda i: (0, 0)),
          pl.BlockSpec((_D, _D), lambda i: (0, 0)),
          pl.BlockSpec((1, _D), lambda i: (0, 0)),
      ],
      out_specs=pl.BlockSpec((_BR, _D), lambda i: (i, 0)),
      out_shape=jax.ShapeDtypeStruct((_NP, _D), jnp.float32),
  )


_dense_relu = _make_dense(True)
_dense = _make_dense(False)


def _scores_body(h_hbm, si_hbm, di_hbm, nsi_hbm, ndi_hbm,
                 pos_hbm, neg_hbm, ia0, ib0, ia1, ib1,
                 hs0, hd0, hs1, hd1, sbuf, semi0, semi1, semg0, semg1, h2s):
  c = lax.axis_index("c")
  s = lax.axis_index("s")
  w = c * _NS + s
  lanes = lax.iota(jnp.int32, _L)

  for k in range(_NRC):
    r0 = s * _RPT + k * _RC
    pltpu.sync_copy(h_hbm.at[pl.ds(r0, _RC)], hs0)
    pltpu.sync_copy(hs0, h2s.at[pl.ds(r0, _RC)])
  plsc.subcore_barrier()

  def start_idx(a_h, b_h, j, ia, ib, sem):
    pltpu.async_copy(a_h.at[pl.ds(w * _EPT + j * _CH, _CH)], ia, sem)
    pltpu.async_copy(b_h.at[pl.ds(w * _EPT + j * _CH, _CH)], ib, sem)

  def wait_idx(a_h, b_h, j, ia, ib, sem):
    pltpu.make_async_copy(a_h.at[pl.ds(w * _EPT + j * _CH, _CH)], ia, sem).wait()
    pltpu.make_async_copy(b_h.at[pl.ds(w * _EPT + j * _CH, _CH)], ib, sem).wait()

  def start_gather(ia, ib, hs, hd, sem):
    pltpu.async_copy(h2s.at[ia], hs, sem)
    pltpu.async_copy(h2s.at[ib], hd, sem)

  def wait_gather(ia, ib, hs, hd, sem):
    pltpu.make_async_copy(h2s.at[ia], hs, sem).wait()
    pltpu.make_async_copy(h2s.at[ib], hd, sem).wait()

  def compute(j, hs, hd, out_h):
    @pl.loop(0, _CH // _L)
    def _(g):
      erow = g * _L + lanes
      zero = jnp.zeros((_L,), jnp.float32)

      @pl.loop(0, _D // 32, init_carry=(zero,) * 8)
      def accs(t, carry):
        new = list(carry)
        base = t * 32
        for u in range(32):
          col = jnp.bitwise_and(base + (u + lanes), _D - 1)
          new[u % 8] = new[u % 8] + (plsc.load_gather(hs, (erow, col))
                                     * plsc.load_gather(hd, (erow, col)))
        return tuple(new)

      acc = (((accs[0] + accs[1]) + (accs[2] + accs[3]))
             + ((accs[4] + accs[5]) + (accs[6] + accs[7])))
      sbuf[pl.ds(g * _L, _L)] = acc

    pltpu.sync_copy(sbuf, out_h.at[pl.ds(w * _EPT + j * _CH, _CH)])

  for (a_h, b_h, out_h) in ((si_hbm, di_hbm, pos_hbm),
                            (nsi_hbm, ndi_hbm, neg_hbm)):
    start_idx(a_h, b_h, 0, ia0, ib0, semi0)
    wait_idx(a_h, b_h, 0, ia0, ib0, semi0)
    start_gather(ia0, ib0, hs0, hd0, semg0)
    start_idx(a_h, b_h, 1, ia1, ib1, semi1)

    @pl.loop(0, (_NCH - 1) // 2)
    def _(j2):
      j = 2 * j2
      wait_gather(ia0, ib0, hs0, hd0, semg0)
      start_idx(a_h, b_h, j + 2, ia0, ib0, semi0)
      wait_idx(a_h, b_h, j + 1, ia1, ib1, semi1)
      start_gather(ia1, ib1, hs1, hd1, semg1)
      compute(j, hs0, hd0, out_h)
      wait_gather(ia1, ib1, hs1, hd1, semg1)
      start_idx(a_h, b_h, jnp.minimum(j + 3, _NCH - 1), ia1, ib1, semi1)
      wait_idx(a_h, b_h, j + 2, ia0, ib0, semi0)
      start_gather(ia0, ib0, hs0, hd0, semg0)
      compute(j + 1, hs1, hd1, out_h)

    wait_gather(ia0, ib0, hs0, hd0, semg0)
    wait_idx(a_h, b_h, _NCH - 1, ia1, ib1, semi1)
    compute(_NCH - 1, hs0, hd0, out_h)


_scores = pl.kernel(
    _scores_body,
    compiler_params=pltpu.CompilerParams(needs_layout_passes=False),
    out_type=(jax.ShapeDtypeStruct((_E,), jnp.float32),
              jax.ShapeDtypeStruct((_E,), jnp.float32)),
    mesh=_mesh,
    scratch_types=[
        pltpu.VMEM((_CH,), jnp.int32),
        pltpu.VMEM((_CH,), jnp.int32),
        pltpu.VMEM((_CH,), jnp.int32),
        pltpu.VMEM((_CH,), jnp.int32),
        pltpu.VMEM((_CH, _D), jnp.float32),
        pltpu.VMEM((_CH, _D), jnp.float32),
        pltpu.VMEM((_CH, _D), jnp.float32),
        pltpu.VMEM((_CH, _D), jnp.float32),
        pltpu.VMEM((_CH,), jnp.float32),
        pltpu.SemaphoreType.DMA,
        pltpu.SemaphoreType.DMA,
        pltpu.SemaphoreType.DMA,
        pltpu.SemaphoreType.DMA,
        pltpu.VMEM_SHARED((_NP, _D), jnp.float32),
    ],
)


def kernel(x, edge_index, neg_edge_index, W_neigh1, W_self1, b1,
           W_neigh2, W_self2, b2):
  src = edge_index[0].reshape(_NW, _NCH, _CH)
  dst = edge_index[1].reshape(_NW, _NCH, _CH)
  nsrc = neg_edge_index[0].reshape(_NW, _NCH, _CH)
  ndst = neg_edge_index[1].reshape(_NW, _NCH, _CH)

  xp = jnp.pad(x, ((0, _NP - _N), (0, 0)))
  degp = _deg(dst)
  p1 = _segsum(xp, src, dst)
  h1 = _dense_relu(p1, degp, xp, W_neigh1, W_self1, b1.reshape(1, _D))
  p2 = _segsum(h1, src, dst)
  h2 = _dense(p2, degp, h1, W_neigh2, W_self2, b2.reshape(1, _D))
  pos, neg = _scores(h2, edge_index[0], edge_index[1],
                     neg_edge_index[0], neg_edge_index[1])
  return pos.reshape(_E, 1), neg.reshape(_E, 1)


# speedup vs baseline: 3.3726x; 1.0605x over previous
import functools

import jax
import jax.numpy as jnp
from jax import lax
from jax.experimental import pallas as pl
from jax.experimental.pallas import tpu as pltpu
from jax.experimental.pallas import tpu_sc as plsc

_N = 10000
_NP = 10240
_D = 128
_E = 320000
_NC = 2
_NS = 16
_NW = _NC * _NS
_EPT = _E // _NW
_CH = 80
_NCH = _EPT // _CH
_RPT = _NP // _NS
_RC = 80
_NRC = _RPT // _RC
_DW = 16
_L = 16

_mesh = plsc.VectorSubcoreMesh(core_axis_name="c", subcore_axis_name="s")


def _zero_rows(ref, nrows, width):
  z = jnp.zeros((_L,), jnp.float32)

  @pl.loop(0, nrows)
  def _(r):
    for p in range(width // _L):
      ref[r, pl.ds(p * _L, _L)] = z


def _fill_ones(ref, nrows, width):
  o = jnp.ones((_L,), jnp.float32)

  @pl.loop(0, nrows)
  def _(r):
    for p in range(width // _L):
      ref[r, pl.ds(p * _L, _L)] = o


def _segsum_body(x_hbm, src_hbm, dst_hbm, out_hbm,
                 idx_s, idxd0, idxd1, rows0, rows1,
                 semg0, semg1, sems0, sems1, acc):
  c = lax.axis_index("c")
  s = lax.axis_index("s")
  w = c * _NS + s

  pltpu.sync_copy(src_hbm.at[w], idx_s)

  _zero_rows(rows0, _CH, _D)
  for k in range(_NRC):
    pltpu.sync_copy(rows0, acc.at[pl.ds(s * _RPT + k * _RC, _RC)])
  plsc.subcore_barrier()

  def load_idxd(j, idxd):
    pltpu.sync_copy(dst_hbm.at[w, j], idxd)

  def start_gather(j, rows, sem):
    pltpu.async_copy(x_hbm.at[idx_s.at[j]], rows, sem)

  def wait_gather(j, rows, sem):
    pltpu.make_async_copy(x_hbm.at[idx_s.at[j]], rows, sem).wait()

  def start_scatter(rows, idxd, sem):
    pltpu.async_copy(rows, acc.at[idxd], sem, add=True)

  def wait_scatter(rows, idxd, sem):
    pltpu.make_async_copy(rows, acc.at[idxd], sem).wait()

  load_idxd(0, idxd0)
  start_gather(0, rows0, semg0)

  @pl.loop(0, (_NCH - 1) // 2)
  def _(j2):
    j = 2 * j2
    wait_gather(j, rows0, semg0)
    start_scatter(rows0, idxd0, sems0)

    @pl.when(j2 > 0)
    def _():
      wait_scatter(rows1, idxd1, sems1)
    load_idxd(j + 1, idxd1)
    start_gather(j + 1, rows1, semg1)
    wait_gather(j + 1, rows1, semg1)
    start_scatter(rows1, idxd1, sems1)
    wait_scatter(rows0, idxd0, sems0)
    load_idxd(j + 2, idxd0)
    start_gather(j + 2, rows0, semg0)

  wait_gather(_NCH - 1, rows0, semg0)
  start_scatter(rows0, idxd0, sems0)
  wait_scatter(rows1, idxd1, sems1)
  wait_scatter(rows0, idxd0, sems0)

  plsc.subcore_barrier()

  for k in range(_NRC):
    r0 = s * _RPT + k * _RC
    pltpu.sync_copy(acc.at[pl.ds(r0, _RC)], out_hbm.at[c, pl.ds(r0, _RC)])


_segsum = pl.kernel(
    _segsum_body,
    out_type=jax.ShapeDtypeStruct((_NC, _NP, _D), jnp.float32),
    mesh=_mesh,
    scratch_types=[
        pltpu.VMEM((_NCH, _CH), jnp.int32),
        pltpu.VMEM((_CH,), jnp.int32),
        pltpu.VMEM((_CH,), jnp.int32),
        pltpu.VMEM((_CH, _D), jnp.float32),
        pltpu.VMEM((_CH, _D), jnp.float32),
        pltpu.SemaphoreType.DMA,
        pltpu.SemaphoreType.DMA,
        pltpu.SemaphoreType.DMA,
        pltpu.SemaphoreType.DMA,
        pltpu.VMEM_SHARED((_NP, _D), jnp.float32),
    ],
)


def _deg_body(dst_hbm, deg_hbm, idxd0, idxd1, buf, sems0, sems1, dacc):
  c = lax.axis_index("c")
  s = lax.axis_index("s")
  w = c * _NS + s

  _zero_rows(buf, _CH, _D)
  for k in range(_NRC):
    pltpu.sync_copy(buf, dacc.at[pl.ds(s * _RPT + k * _RC, _RC)])
  _fill_ones(buf, _CH, _D)
  plsc.subcore_barrier()

  def start_scatter(idxd, sem):
    pltpu.async_copy(buf, dacc.at[idxd], sem, add=True)

  def wait_scatter(idxd, sem):
    pltpu.make_async_copy(buf, dacc.at[idxd], sem).wait()

  pltpu.sync_copy(dst_hbm.at[w, 0], idxd0)
  start_scatter(idxd0, sems0)

  @pl.loop(0, (_NCH - 1) // 2)
  def _(j2):
    j = 2 * j2

    @pl.when(j2 > 0)
    def _():
      wait_scatter(idxd1, sems1)
    pltpu.sync_copy(dst_hbm.at[w, j + 1], idxd1)
    start_scatter(idxd1, sems1)
    wait_scatter(idxd0, sems0)
    pltpu.sync_copy(dst_hbm.at[w, j + 2], idxd0)
    start_scatter(idxd0, sems0)

  wait_scatter(idxd1, sems1)
  wait_scatter(idxd0, sems0)
  plsc.subcore_barrier()
  for k in range(_NRC):
    r0 = s * _RPT + k * _RC
    pltpu.sync_copy(dacc.at[pl.ds(r0, _RC)], deg_hbm.at[c, pl.ds(r0, _RC)])


_deg = pl.kernel(
    _deg_body,
    out_type=jax.ShapeDtypeStruct((_NC, _NP, _D), jnp.float32),
    mesh=_mesh,
    scratch_types=[
        pltpu.VMEM((_CH,), jnp.int32),
        pltpu.VMEM((_CH,), jnp.int32),
        pltpu.VMEM((_CH, _D), jnp.float32),
        pltpu.SemaphoreType.DMA,
        pltpu.SemaphoreType.DMA,
        pltpu.VMEM_SHARED((_NP, _D), jnp.float32),
    ],
)


def _dense_body(relu, p_ref, d_ref, x_ref, wn_ref, ws_ref, b_ref, o_ref):
  agg = p_ref[0] + p_ref[1]
  deg = d_ref[0, :, 0:1] + d_ref[1, :, 0:1]
  mean = agg / jnp.maximum(deg, 1.0)
  h = (jnp.dot(mean, wn_ref[...], preferred_element_type=jnp.float32)
       + jnp.dot(x_ref[...], ws_ref[...], preferred_element_type=jnp.float32)
       + b_ref[...])
  o_ref[...] = jnp.maximum(h, 0.0) if relu else h


_BR = 1024


def _make_dense(relu):
  return pl.pallas_call(
      functools.partial(_dense_body, relu),
      grid=(_NP // _BR,),
      in_specs=[
          pl.BlockSpec((_NC, _BR, _D), lambda i: (0, i, 0)),
          pl.BlockSpec((_NC, _BR, _D), lambda i: (0, i, 0)),
          pl.BlockSpec((_BR, _D), lambda i: (i, 0)),
          pl.BlockSpec((_D, _D), lambda i: (0, 0)),
          pl.BlockSpec((_D, _D), lambda i: (0, 0)),
          pl.BlockSpec((1, _D), lambda i: (0, 0)),
      ],
      out_specs=pl.BlockSpec((_BR, _D), lambda i: (i, 0)),
      out_shape=jax.ShapeDtypeStruct((_NP, _D), jnp.float32),
  )


_dense_relu = _make_dense(True)
_dense = _make_dense(False)


def _scores_body(h_hbm, si_hbm, di_hbm, nsi_hbm, ndi_hbm,
                 pos_hbm, neg_hbm, ia0, ib0, ia1, ib1,
                 hs0, hd0, hs1, hd1, sbuf, semi0, semi1, semg0, semg1, h2s):
  c = lax.axis_index("c")
  s = lax.axis_index("s")
  w = c * _NS + s
  lanes = lax.iota(jnp.int32, _L)

  for k in range(_NRC):
    r0 = s * _RPT + k * _RC
    pltpu.sync_copy(h_hbm.at[pl.ds(r0, _RC)], hs0)
    pltpu.sync_copy(hs0, h2s.at[pl.ds(r0, _RC)])
  plsc.subcore_barrier()

  def start_idx(a_h, b_h, j, ia, ib, sem):
    pltpu.async_copy(a_h.at[pl.ds(w * _EPT + j * _CH, _CH)], ia, sem)
    pltpu.async_copy(b_h.at[pl.ds(w * _EPT + j * _CH, _CH)], ib, sem)

  def wait_idx(a_h, b_h, j, ia, ib, sem):
    pltpu.make_async_copy(a_h.at[pl.ds(w * _EPT + j * _CH, _CH)], ia, sem).wait()
    pltpu.make_async_copy(b_h.at[pl.ds(w * _EPT + j * _CH, _CH)], ib, sem).wait()

  def start_gather(ia, ib, hs, hd, sem):
    pltpu.async_copy(h2s.at[ia], hs, sem)
    pltpu.async_copy(h2s.at[ib], hd, sem)

  def wait_gather(ia, ib, hs, hd, sem):
    pltpu.make_async_copy(h2s.at[ia], hs, sem).wait()
    pltpu.make_async_copy(h2s.at[ib], hd, sem).wait()

  def compute(j, hs, hd, out_h):
    @pl.loop(0, _CH // _L)
    def _(g):
      erow = g * _L + lanes
      zero = jnp.zeros((_L,), jnp.float32)

      @pl.loop(0, _D // 32, init_carry=(zero,) * 8)
      def accs(t, carry):
        new = list(carry)
        base = t * 32
        for u in range(32):
          col = jnp.bitwise_and(base + (u + lanes), _D - 1)
          new[u % 8] = new[u % 8] + (plsc.load_gather(hs, (erow, col))
                                     * plsc.load_gather(hd, (erow, col)))
        return tuple(new)

      acc = (((accs[0] + accs[1]) + (accs[2] + accs[3]))
             + ((accs[4] + accs[5]) + (accs[6] + accs[7])))
      sbuf[pl.ds(g * _L, _L)] = acc

    pltpu.sync_copy(sbuf, out_h.at[pl.ds(w * _EPT + j * _CH, _CH)])

  for (a_h, b_h, out_h) in ((si_hbm, di_hbm, pos_hbm),
                            (nsi_hbm, ndi_hbm, neg_hbm)):
    start_idx(a_h, b_h, 0, ia0, ib0, semi0)
    wait_idx(a_h, b_h, 0, ia0, ib0, semi0)
    start_gather(ia0, ib0, hs0, hd0, semg0)
    start_idx(a_h, b_h, 1, ia1, ib1, semi1)

    @pl.loop(0, (_NCH - 1) // 2)
    def _(j2):
      j = 2 * j2
      wait_gather(ia0, ib0, hs0, hd0, semg0)
      start_idx(a_h, b_h, j + 2, ia0, ib0, semi0)
      wait_idx(a_h, b_h, j + 1, ia1, ib1, semi1)
      start_gather(ia1, ib1, hs1, hd1, semg1)
      compute(j, hs0, hd0, out_h)
      wait_gather(ia1, ib1, hs1, hd1, semg1)
      start_idx(a_h, b_h, jnp.minimum(j + 3, _NCH - 1), ia1, ib1, semi1)
      wait_idx(a_h, b_h, j + 2, ia0, ib0, semi0)
      start_gather(ia0, ib0, hs0, hd0, semg0)
      compute(j + 1, hs1, hd1, out_h)

    wait_gather(ia0, ib0, hs0, hd0, semg0)
    wait_idx(a_h, b_h, _NCH - 1, ia1, ib1, semi1)
    compute(_NCH - 1, hs0, hd0, out_h)


_scores = pl.kernel(
    _scores_body,
    compiler_params=pltpu.CompilerParams(needs_layout_passes=False),
    out_type=(jax.ShapeDtypeStruct((_E,), jnp.float32),
              jax.ShapeDtypeStruct((_E,), jnp.float32)),
    mesh=_mesh,
    scratch_types=[
        pltpu.VMEM((_CH,), jnp.int32),
        pltpu.VMEM((_CH,), jnp.int32),
        pltpu.VMEM((_CH,), jnp.int32),
        pltpu.VMEM((_CH,), jnp.int32),
        pltpu.VMEM((_CH, _D), jnp.float32),
        pltpu.VMEM((_CH, _D), jnp.float32),
        pltpu.VMEM((_CH, _D), jnp.float32),
        pltpu.VMEM((_CH, _D), jnp.float32),
        pltpu.VMEM((_CH,), jnp.float32),
        pltpu.SemaphoreType.DMA,
        pltpu.SemaphoreType.DMA,
        pltpu.SemaphoreType.DMA,
        pltpu.SemaphoreType.DMA,
        pltpu.VMEM_SHARED((_NP, _D), jnp.float32),
    ],
)


def kernel(x, edge_index, neg_edge_index, W_neigh1, W_self1, b1,
           W_neigh2, W_self2, b2):
  src = edge_index[0].reshape(_NW, _NCH, _CH)
  dst = edge_index[1].reshape(_NW, _NCH, _CH)
  nsrc = neg_edge_index[0].reshape(_NW, _NCH, _CH)
  ndst = neg_edge_index[1].reshape(_NW, _NCH, _CH)

  xp = jnp.pad(x, ((0, _NP - _N), (0, 0)))
  degp = _deg(dst)
  p1 = _segsum(xp, src, dst)
  h1 = _dense_relu(p1, degp, xp, W_neigh1, W_self1, b1.reshape(1, _D))
  p2 = _segsum(h1, src, dst)
  h2 = _dense(p2, degp, h1, W_neigh2, W_self2, b2.reshape(1, _D))
  pos, neg = _scores(h2, edge_index[0], edge_index[1],
                     neg_edge_index[0], neg_edge_index[1])
  return pos.reshape(_E, 1), neg.reshape(_E, 1)
